# Initial kernel scaffold; baseline (speedup 1.0000x reference)
import jax
import jax.numpy as jnp
from jax.experimental import pallas as pl

NUM_LAYERS = 8
NUM_EXPERTS = 64
HIDDEN = 4096
TOP_K = 2
BT = 1024


def _router_block(x_ref, wt_ref, w1_ref, w2_ref, i1_ref, i2_ref):
    x = x_ref[...]
    wt = wt_ref[...]
    logits = jax.lax.dot_general(
        x, wt, (((1,), (0,)), ((), ())),
        preferred_element_type=jnp.float32,
        precision=jax.lax.Precision.HIGHEST,
    )
    lg = logits.reshape(BT, NUM_LAYERS, NUM_EXPERTS)
    iota = jax.lax.broadcasted_iota(jnp.int32, lg.shape, 2)

    m1 = jnp.max(lg, axis=2, keepdims=True)
    i1 = jnp.min(jnp.where(lg == m1, iota, NUM_EXPERTS), axis=2)
    masked = jnp.where(iota == i1[:, :, None], -jnp.inf, lg)
    m2 = jnp.max(masked, axis=2, keepdims=True)
    i2 = jnp.min(jnp.where(masked == m2, iota, NUM_EXPERTS), axis=2)

    g = jnp.exp(m2[:, :, 0] - m1[:, :, 0])
    w1 = 1.0 / (1.0 + g)
    w1_ref[...] = w1
    w2_ref[...] = 1.0 - w1
    i1_ref[...] = i1
    i2_ref[...] = i2


def kernel(hidden_states, router_weights):
    t = hidden_states.shape[0]
    le = NUM_LAYERS * NUM_EXPERTS
    wt = router_weights.reshape(le, HIDDEN).T

    grid = (t // BT,)
    out_shapes = [
        jax.ShapeDtypeStruct((t, NUM_LAYERS), jnp.float32),
        jax.ShapeDtypeStruct((t, NUM_LAYERS), jnp.float32),
        jax.ShapeDtypeStruct((t, NUM_LAYERS), jnp.int32),
        jax.ShapeDtypeStruct((t, NUM_LAYERS), jnp.int32),
    ]
    out_spec = pl.BlockSpec((BT, NUM_LAYERS), lambda i: (i, 0))
    w1, w2, i1, i2 = pl.pallas_call(
        _router_block,
        grid=grid,
        in_specs=[
            pl.BlockSpec((BT, HIDDEN), lambda i: (i, 0)),
            pl.BlockSpec((HIDDEN, le), lambda i: (0, 0)),
        ],
        out_specs=[out_spec, out_spec, out_spec, out_spec],
        out_shape=out_shapes,
    )(hidden_states, wt)

    topk_w = jnp.stack([w1, w2], axis=-1).transpose(1, 0, 2)
    topk_i = jnp.stack([i1, i2], axis=-1).transpose(1, 0, 2)
    return topk_w, topk_i


# speedup vs baseline: 2.2567x; 2.2567x over previous
import jax
import jax.numpy as jnp
from jax.experimental import pallas as pl

NUM_LAYERS = 8
NUM_EXPERTS = 64
HIDDEN = 4096
TOP_K = 2
BT = 1024


def _router_block(x_ref, wt_ref, w1_ref, w2_ref, i1_ref, i2_ref):
    x = x_ref[...]
    wt = wt_ref[...]
    logits = jax.lax.dot_general(
        x, wt, (((1,), (0,)), ((), ())),
        preferred_element_type=jnp.float32,
        precision=jax.lax.Precision.DEFAULT,
    )
    lg = logits.reshape(BT, NUM_LAYERS, NUM_EXPERTS)
    iota = jax.lax.broadcasted_iota(jnp.int32, lg.shape, 2)

    m1 = jnp.max(lg, axis=2, keepdims=True)
    i1 = jnp.min(jnp.where(lg == m1, iota, NUM_EXPERTS), axis=2)
    masked = jnp.where(iota == i1[:, :, None], -jnp.inf, lg)
    m2 = jnp.max(masked, axis=2, keepdims=True)
    i2 = jnp.min(jnp.where(masked == m2, iota, NUM_EXPERTS), axis=2)

    g = jnp.exp(m2[:, :, 0] - m1[:, :, 0])
    w1 = 1.0 / (1.0 + g)
    w1_ref[...] = w1
    w2_ref[...] = 1.0 - w1
    i1_ref[...] = i1
    i2_ref[...] = i2


def kernel(hidden_states, router_weights):
    t = hidden_states.shape[0]
    le = NUM_LAYERS * NUM_EXPERTS
    wt = router_weights.reshape(le, HIDDEN).T

    grid = (t // BT,)
    out_shapes = [
        jax.ShapeDtypeStruct((t, NUM_LAYERS), jnp.float32),
        jax.ShapeDtypeStruct((t, NUM_LAYERS), jnp.float32),
        jax.ShapeDtypeStruct((t, NUM_LAYERS), jnp.int32),
        jax.ShapeDtypeStruct((t, NUM_LAYERS), jnp.int32),
    ]
    out_spec = pl.BlockSpec((BT, NUM_LAYERS), lambda i: (i, 0))
    w1, w2, i1, i2 = pl.pallas_call(
        _router_block,
        grid=grid,
        in_specs=[
            pl.BlockSpec((BT, HIDDEN), lambda i: (i, 0)),
            pl.BlockSpec((HIDDEN, le), lambda i: (0, 0)),
        ],
        out_specs=[out_spec, out_spec, out_spec, out_spec],
        out_shape=out_shapes,
    )(hidden_states, wt)

    topk_w = jnp.stack([w1, w2], axis=-1).transpose(1, 0, 2)
    topk_i = jnp.stack([i1, i2], axis=-1).transpose(1, 0, 2)
    return topk_w, topk_i


# speedup vs baseline: 2.3908x; 1.0594x over previous
import jax
import jax.numpy as jnp
from jax.experimental import pallas as pl

NUM_LAYERS = 8
NUM_EXPERTS = 64
HIDDEN = 4096
TOP_K = 2
BT = 1024


def _router_block(x_ref, wt_ref, w1_ref, w2_ref, i1_ref, i2_ref):
    x = x_ref[...]
    wt = wt_ref[...]
    logits = jax.lax.dot_general(
        x, wt, (((1,), (0,)), ((), ())),
        preferred_element_type=jnp.float32,
        precision=jax.lax.Precision.DEFAULT,
    )
    b = jax.lax.bitcast_convert_type(logits, jnp.int32)
    key = b ^ ((b >> 31) & jnp.int32(0x7FFFFFFF))
    iota = jax.lax.broadcasted_iota(jnp.int32, key.shape, 1) & jnp.int32(63)
    key = (key & jnp.int32(~63)) | (jnp.int32(63) - iota)

    k3 = key.reshape(BT, NUM_LAYERS, NUM_EXPERTS)
    k1 = jnp.max(k3, axis=2)
    k2 = jnp.max(jnp.where(k3 == k1[:, :, None], jnp.int32(-2**31), k3), axis=2)

    i1 = jnp.int32(63) - (k1 & jnp.int32(63))
    i2 = jnp.int32(63) - (k2 & jnp.int32(63))
    v1 = k1 & jnp.int32(~63)
    v2 = k2 & jnp.int32(~63)
    m1 = jax.lax.bitcast_convert_type(v1 ^ ((v1 >> 31) & jnp.int32(0x7FFFFFFF)), jnp.float32)
    m2 = jax.lax.bitcast_convert_type(v2 ^ ((v2 >> 31) & jnp.int32(0x7FFFFFFF)), jnp.float32)

    g = jnp.exp(m2 - m1)
    w1 = 1.0 / (1.0 + g)
    w1_ref[...] = w1
    w2_ref[...] = 1.0 - w1
    i1_ref[...] = i1
    i2_ref[...] = i2


def kernel(hidden_states, router_weights):
    t = hidden_states.shape[0]
    le = NUM_LAYERS * NUM_EXPERTS
    wt = router_weights.reshape(le, HIDDEN).T

    grid = (t // BT,)
    out_shapes = [
        jax.ShapeDtypeStruct((t, NUM_LAYERS), jnp.float32),
        jax.ShapeDtypeStruct((t, NUM_LAYERS), jnp.float32),
        jax.ShapeDtypeStruct((t, NUM_LAYERS), jnp.int32),
        jax.ShapeDtypeStruct((t, NUM_LAYERS), jnp.int32),
    ]
    out_spec = pl.BlockSpec((BT, NUM_LAYERS), lambda i: (i, 0))
    w1, w2, i1, i2 = pl.pallas_call(
        _router_block,
        grid=grid,
        in_specs=[
            pl.BlockSpec((BT, HIDDEN), lambda i: (i, 0)),
            pl.BlockSpec((HIDDEN, le), lambda i: (0, 0)),
        ],
        out_specs=[out_spec, out_spec, out_spec, out_spec],
        out_shape=out_shapes,
    )(hidden_states, wt)

    topk_w = jnp.stack([w1, w2], axis=-1).transpose(1, 0, 2)
    topk_i = jnp.stack([i1, i2], axis=-1).transpose(1, 0, 2)
    return topk_w, topk_i


# speedup vs baseline: 2.9457x; 1.2321x over previous
import jax
import jax.numpy as jnp
from jax.experimental import pallas as pl

NUM_LAYERS = 8
NUM_EXPERTS = 64
HIDDEN = 4096
TOP_K = 2
BT = 256


def _decode(kmax):
    kb = jax.lax.bitcast_convert_type(kmax, jnp.int32)
    smask = kb >> 31
    idx = (kb & jnp.int32(63)) ^ (jnp.int32(63) & ~smask)
    val = jax.lax.bitcast_convert_type(kb & jnp.int32(~63), jnp.float32)
    return idx, val


def _router_block(x_ref, wt_ref, w1_ref, w2_ref, i1_ref, i2_ref):
    x = x_ref[...]
    wt = wt_ref[...]
    logits = jax.lax.dot_general(
        x, wt, (((1,), (0,)), ((), ())),
        preferred_element_type=jnp.float32,
        precision=jax.lax.Precision.DEFAULT,
    )

    b = jax.lax.bitcast_convert_type(logits, jnp.int32)
    e6 = jax.lax.broadcasted_iota(jnp.int32, b.shape, 1) & jnp.int32(63)
    eb = e6 ^ (jnp.int32(63) & ~(b >> 31))
    key = jax.lax.bitcast_convert_type((b & jnp.int32(~63)) | eb, jnp.float32)

    k3 = key.reshape(BT, NUM_LAYERS, NUM_EXPERTS)
    k1 = jnp.max(k3, axis=2)
    k2 = jnp.max(jnp.where(k3 == k1[:, :, None], -jnp.inf, k3), axis=2)

    i1, m1 = _decode(k1)
    i2, m2 = _decode(k2)

    g = jnp.exp(m2 - m1)
    w1 = 1.0 / (1.0 + g)
    w1_ref[...] = w1
    w2_ref[...] = 1.0 - w1
    i1_ref[...] = i1
    i2_ref[...] = i2


def kernel(hidden_states, router_weights):
    t = hidden_states.shape[0]
    le = NUM_LAYERS * NUM_EXPERTS
    wt = router_weights.reshape(le, HIDDEN).T

    grid = (t // BT,)
    out_shapes = [
        jax.ShapeDtypeStruct((t, NUM_LAYERS), jnp.float32),
        jax.ShapeDtypeStruct((t, NUM_LAYERS), jnp.float32),
        jax.ShapeDtypeStruct((t, NUM_LAYERS), jnp.int32),
        jax.ShapeDtypeStruct((t, NUM_LAYERS), jnp.int32),
    ]
    out_spec = pl.BlockSpec((BT, NUM_LAYERS), lambda i: (i, 0))
    w1, w2, i1, i2 = pl.pallas_call(
        _router_block,
        grid=grid,
        in_specs=[
            pl.BlockSpec((BT, HIDDEN), lambda i: (i, 0)),
            pl.BlockSpec((HIDDEN, le), lambda i: (0, 0)),
        ],
        out_specs=[out_spec, out_spec, out_spec, out_spec],
        out_shape=out_shapes,
    )(hidden_states, wt)

    topk_w = jnp.stack([w1, w2], axis=-1).transpose(1, 0, 2)
    topk_i = jnp.stack([i1, i2], axis=-1).transpose(1, 0, 2)
    return topk_w, topk_i


# speedup vs baseline: 3.2279x; 1.0958x over previous
import jax
import jax.numpy as jnp
from jax.experimental import pallas as pl

NUM_LAYERS = 8
NUM_EXPERTS = 64
HIDDEN = 4096
TOP_K = 2
BT = 256


def _decode(kmax):
    kb = jax.lax.bitcast_convert_type(kmax, jnp.int32)
    smask = kb >> 31
    idx = (kb & jnp.int32(63)) ^ (jnp.int32(63) & ~smask)
    val = jax.lax.bitcast_convert_type(kb & jnp.int32(~63), jnp.float32)
    return idx, val


def _router_block(x_ref, wt_ref, w1_ref, w2_ref, i1_ref, i2_ref):
    x = x_ref[...]
    wt = wt_ref[...]
    logits = jax.lax.dot_general(
        x, wt, (((1,), (0,)), ((), ())),
        preferred_element_type=jnp.float32,
        precision=jax.lax.Precision.DEFAULT,
    )

    b = jax.lax.bitcast_convert_type(logits, jnp.int32)
    e6 = jax.lax.broadcasted_iota(jnp.int32, b.shape, 1) >> 3
    eb = e6 ^ (jnp.int32(63) & ~(b >> 31))
    key = jax.lax.bitcast_convert_type((b & jnp.int32(~63)) | eb, jnp.float32)

    n = NUM_LAYERS * NUM_EXPERTS // 2
    m1 = jnp.maximum(key[:, :n], key[:, n:])
    m2 = jnp.minimum(key[:, :n], key[:, n:])
    n //= 2
    while n >= NUM_LAYERS:
        a1, b1 = m1[:, :n], m1[:, n:]
        a2, b2 = m2[:, :n], m2[:, n:]
        m1 = jnp.maximum(a1, b1)
        m2 = jnp.maximum(jnp.minimum(a1, b1), jnp.maximum(a2, b2))
        n //= 2

    i1, v1 = _decode(m1)
    i2, v2 = _decode(m2)

    g = jnp.exp(v2 - v1)
    w1 = 1.0 / (1.0 + g)
    w1_ref[...] = w1
    w2_ref[...] = 1.0 - w1
    i1_ref[...] = i1
    i2_ref[...] = i2


def kernel(hidden_states, router_weights):
    t = hidden_states.shape[0]
    le = NUM_LAYERS * NUM_EXPERTS
    wt = router_weights.transpose(2, 1, 0).reshape(HIDDEN, le)

    grid = (t // BT,)
    out_shapes = [
        jax.ShapeDtypeStruct((t, NUM_LAYERS), jnp.float32),
        jax.ShapeDtypeStruct((t, NUM_LAYERS), jnp.float32),
        jax.ShapeDtypeStruct((t, NUM_LAYERS), jnp.int32),
        jax.ShapeDtypeStruct((t, NUM_LAYERS), jnp.int32),
    ]
    out_spec = pl.BlockSpec((BT, NUM_LAYERS), lambda i: (i, 0))
    w1, w2, i1, i2 = pl.pallas_call(
        _router_block,
        grid=grid,
        in_specs=[
            pl.BlockSpec((BT, HIDDEN), lambda i: (i, 0)),
            pl.BlockSpec((HIDDEN, le), lambda i: (0, 0)),
        ],
        out_specs=[out_spec, out_spec, out_spec, out_spec],
        out_shape=out_shapes,
    )(hidden_states, wt)

    topk_w = jnp.stack([w1, w2], axis=-1).transpose(1, 0, 2)
    topk_i = jnp.stack([i1, i2], axis=-1).transpose(1, 0, 2)
    return topk_w, topk_i


# speedup vs baseline: 3.6195x; 1.1213x over previous
import jax
import jax.numpy as jnp
from jax.experimental import pallas as pl

NUM_LAYERS = 8
NUM_EXPERTS = 64
HIDDEN = 4096
TOP_K = 2
BT = 512


def _decode(kmax):
    kb = jax.lax.bitcast_convert_type(kmax, jnp.int32)
    smask = kb >> 31
    idx = (kb & jnp.int32(63)) ^ (jnp.int32(63) & ~smask)
    val = jax.lax.bitcast_convert_type(kb & jnp.int32(~63), jnp.float32)
    return idx, val


def _router_block(x_ref, wt_ref, w1_ref, w2_ref, i1_ref, i2_ref):
    x = x_ref[...]
    wt = wt_ref[...]
    logits = jax.lax.dot_general(
        x, wt, (((1,), (0,)), ((), ())),
        preferred_element_type=jnp.float32,
        precision=jax.lax.Precision.DEFAULT,
    )

    b = jax.lax.bitcast_convert_type(logits, jnp.int32)
    e6 = jax.lax.broadcasted_iota(jnp.int32, b.shape, 1) >> 3
    eb = e6 ^ (jnp.int32(63) & ~(b >> 31))
    key = jax.lax.bitcast_convert_type((b & jnp.int32(~63)) | eb, jnp.float32)

    n = NUM_LAYERS * NUM_EXPERTS // 2
    m1 = jnp.maximum(key[:, :n], key[:, n:])
    m2 = jnp.minimum(key[:, :n], key[:, n:])
    n //= 2
    while n >= NUM_LAYERS:
        a1, b1 = m1[:, :n], m1[:, n:]
        a2, b2 = m2[:, :n], m2[:, n:]
        m1 = jnp.maximum(a1, b1)
        m2 = jnp.maximum(jnp.minimum(a1, b1), jnp.maximum(a2, b2))
        n //= 2

    i1, v1 = _decode(m1)
    i2, v2 = _decode(m2)

    g = jnp.exp(v2 - v1)
    w1 = 1.0 / (1.0 + g)
    w1_ref[...] = w1
    w2_ref[...] = 1.0 - w1
    i1_ref[...] = i1
    i2_ref[...] = i2


def kernel(hidden_states, router_weights):
    t = hidden_states.shape[0]
    le = NUM_LAYERS * NUM_EXPERTS
    wt = router_weights.transpose(2, 1, 0).reshape(HIDDEN, le)

    grid = (t // BT,)
    out_shapes = [
        jax.ShapeDtypeStruct((t, NUM_LAYERS), jnp.float32),
        jax.ShapeDtypeStruct((t, NUM_LAYERS), jnp.float32),
        jax.ShapeDtypeStruct((t, NUM_LAYERS), jnp.int32),
        jax.ShapeDtypeStruct((t, NUM_LAYERS), jnp.int32),
    ]
    out_spec = pl.BlockSpec((BT, NUM_LAYERS), lambda i: (i, 0))
    w1, w2, i1, i2 = pl.pallas_call(
        _router_block,
        grid=grid,
        in_specs=[
            pl.BlockSpec((BT, HIDDEN), lambda i: (i, 0)),
            pl.BlockSpec((HIDDEN, le), lambda i: (0, 0)),
        ],
        out_specs=[out_spec, out_spec, out_spec, out_spec],
        out_shape=out_shapes,
    )(hidden_states, wt)

    topk_w = jnp.stack([w1, w2], axis=-1).transpose(1, 0, 2)
    topk_i = jnp.stack([i1, i2], axis=-1).transpose(1, 0, 2)
    return topk_w, topk_i


# speedup vs baseline: 3.7883x; 1.0467x over previous
import jax
import jax.numpy as jnp
from jax.experimental import pallas as pl

NUM_LAYERS = 8
NUM_EXPERTS = 64
HIDDEN = 4096
TOP_K = 2
BT = 512


def _decode(kmax):
    kb = jax.lax.bitcast_convert_type(kmax, jnp.int32)
    smask = kb >> 31
    idx = (kb & jnp.int32(63)) ^ (jnp.int32(63) & ~smask)
    val = jax.lax.bitcast_convert_type(kb & jnp.int32(~63), jnp.float32)
    return idx, val


def _router_block(x_ref, wt_ref, w1_ref, w2_ref, i1_ref, i2_ref):
    x = x_ref[...]
    wt = wt_ref[...]
    logits = jax.lax.dot_general(
        x, wt, (((1,), (1,)), ((), ())),
        preferred_element_type=jnp.float32,
        precision=jax.lax.Precision.DEFAULT,
    )

    b = jax.lax.bitcast_convert_type(logits, jnp.int32)
    e6 = jax.lax.broadcasted_iota(jnp.int32, b.shape, 1) >> 3
    eb = e6 ^ (jnp.int32(63) & ~(b >> 31))
    key = jax.lax.bitcast_convert_type((b & jnp.int32(~63)) | eb, jnp.float32)

    n = NUM_LAYERS * NUM_EXPERTS // 2
    m1 = jnp.maximum(key[:, :n], key[:, n:])
    m2 = jnp.minimum(key[:, :n], key[:, n:])
    n //= 2
    while n >= NUM_LAYERS:
        a1, b1 = m1[:, :n], m1[:, n:]
        a2, b2 = m2[:, :n], m2[:, n:]
        m1 = jnp.maximum(a1, b1)
        m2 = jnp.maximum(jnp.minimum(a1, b1), jnp.maximum(a2, b2))
        n //= 2

    i1, v1 = _decode(m1)
    i2, v2 = _decode(m2)

    g = jnp.exp(v2 - v1)
    w1 = 1.0 / (1.0 + g)
    w1_ref[...] = w1
    w2_ref[...] = 1.0 - w1
    i1_ref[...] = i1
    i2_ref[...] = i2


def kernel(hidden_states, router_weights):
    t = hidden_states.shape[0]
    le = NUM_LAYERS * NUM_EXPERTS
    wt = router_weights.transpose(1, 0, 2).reshape(le, HIDDEN)

    grid = (t // BT,)
    out_shapes = [
        jax.ShapeDtypeStruct((t, NUM_LAYERS), jnp.float32),
        jax.ShapeDtypeStruct((t, NUM_LAYERS), jnp.float32),
        jax.ShapeDtypeStruct((t, NUM_LAYERS), jnp.int32),
        jax.ShapeDtypeStruct((t, NUM_LAYERS), jnp.int32),
    ]
    out_spec = pl.BlockSpec((BT, NUM_LAYERS), lambda i: (i, 0))
    w1, w2, i1, i2 = pl.pallas_call(
        _router_block,
        grid=grid,
        in_specs=[
            pl.BlockSpec((BT, HIDDEN), lambda i: (i, 0)),
            pl.BlockSpec((le, HIDDEN), lambda i: (0, 0)),
        ],
        out_specs=[out_spec, out_spec, out_spec, out_spec],
        out_shape=out_shapes,
    )(hidden_states, wt)

    topk_w = jnp.stack([w1, w2], axis=-1).transpose(1, 0, 2)
    topk_i = jnp.stack([i1, i2], axis=-1).transpose(1, 0, 2)
    return topk_w, topk_i
